# baseline (speedup 1.0000x reference)
import jax, jax.numpy as jnp
import numpy as np
from jax.experimental import pallas as pl

H = 128
bf = jnp.bfloat16
f32 = jnp.float32
E = 160000
BE = 4000

def bdot(a, b):
    return jax.lax.dot(a.astype(bf), b.astype(bf), preferred_element_type=f32)


def _edge_mlp_body(hs, hd, ef, w1, b1, w2, b2, out):
    mi = jnp.concatenate([hs[:], hd[:], ef[:]], axis=-1)
    m = jax.nn.relu(jax.lax.dot(mi.astype(bf), w1[:].astype(bf),
                                preferred_element_type=f32) + b1[:])
    out[:] = jax.lax.dot(m.astype(bf), w2[:].astype(bf),
                         preferred_element_type=f32) + b2[:]


_I0 = np.int32(0)

_edge_mlp = pl.pallas_call(
    _edge_mlp_body,
    grid=(E // BE,),
    in_specs=[pl.BlockSpec((BE, H), lambda i: (i, _I0)),
              pl.BlockSpec((BE, H), lambda i: (i, _I0)),
              pl.BlockSpec((BE, 4), lambda i: (i, _I0)),
              pl.BlockSpec((2 * H + 4, H), lambda i: (_I0, _I0)),
              pl.BlockSpec((1, H), lambda i: (_I0, _I0)),
              pl.BlockSpec((H, H), lambda i: (_I0, _I0)),
              pl.BlockSpec((1, H), lambda i: (_I0, _I0))],
    out_specs=pl.BlockSpec((BE, H), lambda i: (i, _I0)),
    out_shape=jax.ShapeDtypeStruct((E, H), f32),
)


def kernel(node_feats, edge_index, edge_feats, W_in, b_in, W_m1, b_m1, W_m2, b_m2, W_z, b_z, W_r, b_r, W_h, b_h, ln_s, ln_b, W_c1, b_c1, W_c2, b_c2, W_p1, b_p1, W_p2, b_p2):
    n = node_feats.shape[0]
    h = bdot(node_feats, W_in) + b_in
    src = edge_index[0]
    dst = edge_index[1]
    for t in range(W_m1.shape[0]):
        msg = _edge_mlp(h[src], h[dst], edge_feats, W_m1[t], b_m1[t].reshape(1, H),
                        W_m2[t], b_m2[t].reshape(1, H))
        agg = jnp.zeros((n, h.shape[1]), dtype=h.dtype).at[dst].add(msg)
        gate_in = jnp.concatenate([h, agg], axis=-1)
        z = jax.nn.sigmoid(bdot(gate_in, W_z[t]) + b_z[t])
        r = jax.nn.sigmoid(bdot(gate_in, W_r[t]) + b_r[t])
        h_new = jnp.tanh(bdot(jnp.concatenate([r * h, agg], axis=-1), W_h[t]) + b_h[t])
        hu = (1 - z) * h + z * h_new
        mean = jnp.mean(hu, axis=-1, keepdims=True)
        var = jnp.var(hu, axis=-1, keepdims=True)
        h = (hu - mean) / jnp.sqrt(var + 1e-6) * ln_s[t] + ln_b[t]
    cls = bdot(jax.nn.relu(bdot(h, W_c1) + b_c1), W_c2) + b_c2
    prob = jax.nn.sigmoid(bdot(jax.nn.relu(bdot(h, W_p1) + b_p1), W_p2) + b_p2).squeeze(-1)
    return (cls, prob)


# speedup vs baseline: 1.0995x; 1.0995x over previous
import jax
import jax.numpy as jnp
import numpy as np
from jax import lax
from jax.experimental import pallas as pl
from jax.experimental.pallas import tpu as pltpu
from jax.experimental.pallas import tpu_sc as plsc

N = 10000
E = 160000
H = 128
T = 6

NC = 2
NS = 16
NW = NC * NS

CHUNK = 128
NCHUNK = E // CHUNK
CPW = 40
NCHUNK_PAD = NW * CPW
E_PAD = NCHUNK_PAD * CHUNK

bf = jnp.bfloat16
f32 = jnp.float32


def bdot(a, b):
    return jax.lax.dot(a.astype(bf), b.astype(bf), preferred_element_type=f32)


_SC_MESH = plsc.VectorSubcoreMesh(
    core_axis_name="c", subcore_axis_name="s", num_cores=NC, num_subcores=NS)


def _gather_body(h_tab, src2d, dst2d, hs_out, hd_out,
                 idx_s, idx_d, bs0, bd0, bs1, bd1, sem):
    cid = lax.axis_index("c")
    sid = lax.axis_index("s")
    wid = cid * NS + sid
    base = wid * CPW
    pltpu.sync_copy(src2d.at[pl.ds(base, CPW)], idx_s)
    pltpu.sync_copy(dst2d.at[pl.ds(base, CPW)], idx_d)

    def pair(j2, carry):
        j = j2 * 2
        c0 = base + j
        c1 = c0 + 1
        g0 = pltpu.async_copy(h_tab.at[idx_s.at[j]], bs0, sem)
        g1 = pltpu.async_copy(h_tab.at[idx_d.at[j]], bd0, sem)
        g2 = pltpu.async_copy(h_tab.at[idx_s.at[j + 1]], bs1, sem)
        g3 = pltpu.async_copy(h_tab.at[idx_d.at[j + 1]], bd1, sem)
        g0.wait()
        g1.wait()
        g2.wait()
        g3.wait()

        @pl.when(c0 < NCHUNK)
        def _():
            pltpu.sync_copy(bs0, hs_out.at[pl.ds(c0 * CHUNK, CHUNK)])
            pltpu.sync_copy(bd0, hd_out.at[pl.ds(c0 * CHUNK, CHUNK)])

        @pl.when(c1 < NCHUNK)
        def _():
            pltpu.sync_copy(bs1, hs_out.at[pl.ds(c1 * CHUNK, CHUNK)])
            pltpu.sync_copy(bd1, hd_out.at[pl.ds(c1 * CHUNK, CHUNK)])

        return carry

    lax.fori_loop(jnp.int32(0), jnp.int32(CPW // 2), pair, jnp.int32(0))


_gather_kernel = pl.kernel(
    _gather_body,
    out_type=[jax.ShapeDtypeStruct((E, H), f32),
              jax.ShapeDtypeStruct((E, H), f32)],
    mesh=_SC_MESH,
    scratch_types=[
        pltpu.VMEM((CPW, CHUNK), jnp.int32),
        pltpu.VMEM((CPW, CHUNK), jnp.int32),
        pltpu.VMEM((CHUNK, H), f32),
        pltpu.VMEM((CHUNK, H), f32),
        pltpu.VMEM((CHUNK, H), f32),
        pltpu.VMEM((CHUNK, H), f32),
        pltpu.SemaphoreType.DMA,
    ],
    compiler_params=pltpu.CompilerParams(use_tc_tiling_on_sc=False),
)


BE = 4000
_I0 = np.int32(0)


def _edge_mlp_body(hs, hd, ef, w1, b1, w2, b2, out):
    mi = jnp.concatenate([hs[:], hd[:], ef[:]], axis=-1)
    m = jax.nn.relu(jax.lax.dot(mi.astype(bf), w1[:].astype(bf),
                                preferred_element_type=f32) + b1[:])
    out[:] = jax.lax.dot(m.astype(bf), w2[:].astype(bf),
                         preferred_element_type=f32) + b2[:]


_edge_mlp = pl.pallas_call(
    _edge_mlp_body,
    grid=(E // BE,),
    in_specs=[pl.BlockSpec((BE, H), lambda i: (i, _I0)),
              pl.BlockSpec((BE, H), lambda i: (i, _I0)),
              pl.BlockSpec((BE, 4), lambda i: (i, _I0)),
              pl.BlockSpec((2 * H + 4, H), lambda i: (_I0, _I0)),
              pl.BlockSpec((1, H), lambda i: (_I0, _I0)),
              pl.BlockSpec((H, H), lambda i: (_I0, _I0)),
              pl.BlockSpec((1, H), lambda i: (_I0, _I0))],
    out_specs=pl.BlockSpec((BE, H), lambda i: (i, _I0)),
    out_shape=jax.ShapeDtypeStruct((E, H), f32),
)


def kernel(node_feats, edge_index, edge_feats, W_in, b_in, W_m1, b_m1,
           W_m2, b_m2, W_z, b_z, W_r, b_r, W_h, b_h, ln_s, ln_b,
           W_c1, b_c1, W_c2, b_c2, W_p1, b_p1, W_p2, b_p2):
    n = node_feats.shape[0]
    src = edge_index[0]
    dst = edge_index[1]
    pad_c = E_PAD - E
    src2d = jnp.concatenate(
        [src.astype(jnp.int32), jnp.zeros((pad_c,), jnp.int32)]).reshape(NCHUNK_PAD, CHUNK)
    dst2d = jnp.concatenate(
        [dst.astype(jnp.int32), jnp.zeros((pad_c,), jnp.int32)]).reshape(NCHUNK_PAD, CHUNK)

    h = bdot(node_feats, W_in) + b_in
    for t in range(T):
        hs, hd = _gather_kernel(h, src2d, dst2d)
        msg = _edge_mlp(hs, hd, edge_feats, W_m1[t], b_m1[t].reshape(1, H),
                        W_m2[t], b_m2[t].reshape(1, H))
        agg = jnp.zeros((n, H), dtype=h.dtype).at[dst].add(msg)
        gate_in = jnp.concatenate([h, agg], axis=-1)
        z = jax.nn.sigmoid(bdot(gate_in, W_z[t]) + b_z[t])
        r = jax.nn.sigmoid(bdot(gate_in, W_r[t]) + b_r[t])
        h_new = jnp.tanh(bdot(jnp.concatenate([r * h, agg], axis=-1), W_h[t]) + b_h[t])
        hu = (1 - z) * h + z * h_new
        mean = jnp.mean(hu, axis=-1, keepdims=True)
        var = jnp.var(hu, axis=-1, keepdims=True)
        h = (hu - mean) / jnp.sqrt(var + 1e-6) * ln_s[t] + ln_b[t]
    cls = bdot(jax.nn.relu(bdot(h, W_c1) + b_c1), W_c2) + b_c2
    prob = jax.nn.sigmoid(bdot(jax.nn.relu(bdot(h, W_p1) + b_p1), W_p2) + b_p2).squeeze(-1)
    return (cls, prob)


# speedup vs baseline: 1.1112x; 1.0107x over previous
import jax
import jax.numpy as jnp
import numpy as np
from jax import lax
from jax.experimental import pallas as pl
from jax.experimental.pallas import tpu as pltpu
from jax.experimental.pallas import tpu_sc as plsc

N = 10000
E = 160000
H = 128
T = 6

NC = 2
NS = 16
NW = NC * NS

CHUNK = 64
NCHUNK = E // CHUNK
CPW = 80
NCHUNK_PAD = NW * CPW
E_PAD = NCHUNK_PAD * CHUNK
NSETS = 4
QS = CPW // NSETS

bf = jnp.bfloat16
f32 = jnp.float32


def bdot(a, b):
    return jax.lax.dot(a.astype(bf), b.astype(bf), preferred_element_type=f32)


_SC_MESH = plsc.VectorSubcoreMesh(
    core_axis_name="c", subcore_axis_name="s", num_cores=NC, num_subcores=NS)


def _gather_body(h_tab, src2d, dst2d, hs_out, hd_out,
                 idx_s, idx_d, bs0, bd0, bs1, bd1, bs2, bd2, bs3, bd3,
                 sem0, sem1, sem2, sem3):
    cid = lax.axis_index("c")
    sid = lax.axis_index("s")
    wid = cid * NS + sid
    base = wid * CPW
    pltpu.sync_copy(src2d.at[pl.ds(base, CPW)], idx_s)
    pltpu.sync_copy(dst2d.at[pl.ds(base, CPW)], idx_d)

    sets = ((bs0, bd0, sem0), (bs1, bd1, sem1), (bs2, bd2, sem2), (bs3, bd3, sem3))

    for k in range(NSETS):
        k32 = np.int32(k)
        pltpu.async_copy(h_tab.at[idx_s.at[k32]], sets[k][0], sets[k][2])
        pltpu.async_copy(h_tab.at[idx_d.at[k32]], sets[k][1], sets[k][2])

    def body(q, carry):
        for k in range(NSETS):
            bs, bdd, sem = sets[k]
            k32 = np.int32(k)
            c = base + q * np.int32(NSETS) + k32
            pltpu.make_async_copy(h_tab.at[idx_s.at[k32]], bs, sem).wait()
            pltpu.make_async_copy(h_tab.at[idx_d.at[k32]], bdd, sem).wait()

            @pl.when(c < NCHUNK)
            def _(bs=bs, bdd=bdd, c=c):
                pltpu.sync_copy(bs, hs_out.at[pl.ds(c * CHUNK, CHUNK)])
                pltpu.sync_copy(bdd, hd_out.at[pl.ds(c * CHUNK, CHUNK)])

            @pl.when(q < QS - 1)
            def _(bs=bs, bdd=bdd, sem=sem, k32=k32):
                jn = (q + np.int32(1)) * np.int32(NSETS) + k32
                pltpu.async_copy(h_tab.at[idx_s.at[jn]], bs, sem)
                pltpu.async_copy(h_tab.at[idx_d.at[jn]], bdd, sem)

        return carry

    lax.fori_loop(jnp.int32(0), jnp.int32(QS), body, jnp.int32(0))


_gather_kernel = pl.kernel(
    _gather_body,
    out_type=[jax.ShapeDtypeStruct((E, H), f32),
              jax.ShapeDtypeStruct((E, H), f32)],
    mesh=_SC_MESH,
    scratch_types=[
        pltpu.VMEM((CPW, CHUNK), jnp.int32),
        pltpu.VMEM((CPW, CHUNK), jnp.int32),
        pltpu.VMEM((CHUNK, H), f32),
        pltpu.VMEM((CHUNK, H), f32),
        pltpu.VMEM((CHUNK, H), f32),
        pltpu.VMEM((CHUNK, H), f32),
        pltpu.VMEM((CHUNK, H), f32),
        pltpu.VMEM((CHUNK, H), f32),
        pltpu.VMEM((CHUNK, H), f32),
        pltpu.VMEM((CHUNK, H), f32),
        pltpu.SemaphoreType.DMA,
        pltpu.SemaphoreType.DMA,
        pltpu.SemaphoreType.DMA,
        pltpu.SemaphoreType.DMA,
    ],
    compiler_params=pltpu.CompilerParams(use_tc_tiling_on_sc=False),
)


BE = 4000
_I0 = np.int32(0)


def _edge_mlp_body(hs, hd, ef, w1, b1, w2, b2, out):
    mi = jnp.concatenate([hs[:], hd[:], ef[:]], axis=-1)
    m = jax.nn.relu(jax.lax.dot(mi.astype(bf), w1[:].astype(bf),
                                preferred_element_type=f32) + b1[:])
    out[:] = jax.lax.dot(m.astype(bf), w2[:].astype(bf),
                         preferred_element_type=f32) + b2[:]


_edge_mlp = pl.pallas_call(
    _edge_mlp_body,
    grid=(E // BE,),
    in_specs=[pl.BlockSpec((BE, H), lambda i: (i, _I0)),
              pl.BlockSpec((BE, H), lambda i: (i, _I0)),
              pl.BlockSpec((BE, 4), lambda i: (i, _I0)),
              pl.BlockSpec((2 * H + 4, H), lambda i: (_I0, _I0)),
              pl.BlockSpec((1, H), lambda i: (_I0, _I0)),
              pl.BlockSpec((H, H), lambda i: (_I0, _I0)),
              pl.BlockSpec((1, H), lambda i: (_I0, _I0))],
    out_specs=pl.BlockSpec((BE, H), lambda i: (i, _I0)),
    out_shape=jax.ShapeDtypeStruct((E, H), f32),
)


def kernel(node_feats, edge_index, edge_feats, W_in, b_in, W_m1, b_m1,
           W_m2, b_m2, W_z, b_z, W_r, b_r, W_h, b_h, ln_s, ln_b,
           W_c1, b_c1, W_c2, b_c2, W_p1, b_p1, W_p2, b_p2):
    n = node_feats.shape[0]
    src = edge_index[0]
    dst = edge_index[1]
    pad_c = E_PAD - E
    src2d = jnp.concatenate(
        [src.astype(jnp.int32), jnp.zeros((pad_c,), jnp.int32)]).reshape(NCHUNK_PAD, CHUNK)
    dst2d = jnp.concatenate(
        [dst.astype(jnp.int32), jnp.zeros((pad_c,), jnp.int32)]).reshape(NCHUNK_PAD, CHUNK)

    h = bdot(node_feats, W_in) + b_in
    for t in range(T):
        hs, hd = _gather_kernel(h, src2d, dst2d)
        msg = _edge_mlp(hs, hd, edge_feats, W_m1[t], b_m1[t].reshape(1, H),
                        W_m2[t], b_m2[t].reshape(1, H))
        agg = jnp.zeros((n, H), dtype=h.dtype).at[dst].add(msg)
        gate_in = jnp.concatenate([h, agg], axis=-1)
        z = jax.nn.sigmoid(bdot(gate_in, W_z[t]) + b_z[t])
        r = jax.nn.sigmoid(bdot(gate_in, W_r[t]) + b_r[t])
        h_new = jnp.tanh(bdot(jnp.concatenate([r * h, agg], axis=-1), W_h[t]) + b_h[t])
        hu = (1 - z) * h + z * h_new
        mean = jnp.mean(hu, axis=-1, keepdims=True)
        var = jnp.var(hu, axis=-1, keepdims=True)
        h = (hu - mean) / jnp.sqrt(var + 1e-6) * ln_s[t] + ln_b[t]
    cls = bdot(jax.nn.relu(bdot(h, W_c1) + b_c1), W_c2) + b_c2
    prob = jax.nn.sigmoid(bdot(jax.nn.relu(bdot(h, W_p1) + b_p1), W_p2) + b_p2).squeeze(-1)
    return (cls, prob)
